# baseline (device time: 345528 ns/iter reference)
import jax
import jax.numpy as jnp
from jax import lax
from jax.experimental import pallas as pl
from jax.experimental.pallas import tpu as pltpu

K = 4096
M = 4096
N = 8192
N_HALF = N // 2
M_HALF = M // 2

_BM, _BN, _BK = 1024, 1024, 2048
_NI = M // _BM
_NJ = N_HALF // _BN
_NK = K // _BK
_NOWN = _NI // 2


def _fused(scalars, x_shard, dy_shard):
    def body(
        s_ref,
        x_ref,
        dy_ref,
        stk_ref,
        recvx_ref,
        stage_ref,
        acc_ref,
        b_ref,
        q_ref,
        sendv_ref,
        copy_sems,
        sx_send,
        sx_recv,
        sy_send,
        sy_recv,
    ):
        i = pl.program_id(0)
        j = pl.program_id(1)
        k = pl.program_id(2)
        my_x = s_ref[0]
        my_y = s_ref[1]
        other_x = 1 - my_x
        other_y = 1 - my_y

        @pl.when((i == 0) & (j == 0) & (k == 0))
        def _():
            barrier = pltpu.get_barrier_semaphore()
            pl.semaphore_signal(
                barrier, inc=1, device_id=(other_x, my_y),
                device_id_type=pl.DeviceIdType.MESH,
            )
            pl.semaphore_signal(
                barrier, inc=1, device_id=(my_x, other_y),
                device_id_type=pl.DeviceIdType.MESH,
            )
            pl.semaphore_wait(barrier, 2)

        @pl.when(k == 0)
        def _():
            acc_ref[...] = jnp.zeros_like(acc_ref)

        acc_ref[...] += lax.dot_general(
            x_ref[...],
            dy_ref[...],
            dimension_numbers=(((0,), (0,)), ((), ())),
            preferred_element_type=jnp.float32,
        )

        @pl.when(k == _NK - 1)
        def _finalize():
            rows = pl.ds(i * _BM, _BM)
            rows_own = pl.ds((i - _NOWN) * _BM, _BM)
            cols = pl.ds(j * _BN, _BN)

            @pl.when(i < _NOWN)
            def _send_to_x_neighbor():
                sendv_ref[...] = acc_ref[...].astype(jnp.bfloat16)
                st = pltpu.make_async_copy(
                    sendv_ref, stage_ref.at[rows, cols], copy_sems.at[0]
                )
                st.start()
                st.wait()
                pltpu.make_async_remote_copy(
                    src_ref=stage_ref.at[rows, cols],
                    dst_ref=recvx_ref.at[rows, cols],
                    send_sem=sx_send.at[i, j],
                    recv_sem=sx_recv.at[i, j],
                    device_id=(other_x, my_y),
                    device_id_type=pl.DeviceIdType.MESH,
                ).start()

            @pl.when(i >= _NOWN)
            def _reduce_and_send_y():
                pltpu.make_async_remote_copy(
                    src_ref=stage_ref.at[rows_own, cols],
                    dst_ref=recvx_ref.at[rows_own, cols],
                    send_sem=sx_send.at[i - _NOWN, j],
                    recv_sem=sx_recv.at[i - _NOWN, j],
                    device_id=(other_x, my_y),
                    device_id_type=pl.DeviceIdType.MESH,
                ).wait_recv()
                ld = pltpu.make_async_copy(
                    recvx_ref.at[rows_own, cols], b_ref, copy_sems.at[1]
                )
                ld.start()
                ld.wait()
                flat = (i - _NOWN) * _NJ + j

                @pl.when(flat > 0)
                def _():
                    pr = (flat - 1) // _NJ
                    pj = (flat - 1) % _NJ
                    prows = pl.ds(pr * _BM, _BM)
                    pcols = pl.ds(pj * _BN, _BN)
                    pltpu.make_async_remote_copy(
                        src_ref=q_ref,
                        dst_ref=stk_ref.at[1, prows, pcols],
                        send_sem=sy_send.at[pr, pj],
                        recv_sem=sy_recv.at[pr, pj],
                        device_id=(my_x, other_y),
                        device_id_type=pl.DeviceIdType.MESH,
                    ).wait_send()

                q_ref[...] = (
                    acc_ref[...] + b_ref[...].astype(jnp.float32)
                ).astype(jnp.bfloat16)
                stq = pltpu.make_async_copy(
                    q_ref, stk_ref.at[0, rows_own, cols], copy_sems.at[2]
                )
                stq.start()
                pltpu.make_async_remote_copy(
                    src_ref=q_ref,
                    dst_ref=stk_ref.at[1, rows_own, cols],
                    send_sem=sy_send.at[i - _NOWN, j],
                    recv_sem=sy_recv.at[i - _NOWN, j],
                    device_id=(my_x, other_y),
                    device_id_type=pl.DeviceIdType.MESH,
                ).start()
                stq.wait()

        @pl.when((i == _NI - 1) & (j == _NJ - 1) & (k == _NK - 1))
        def _drain():
            for ii in range(_NOWN):
                for jj in range(_NJ):
                    rs = pl.ds(ii * _BM, _BM)
                    cs = pl.ds(jj * _BN, _BN)
                    pltpu.make_async_remote_copy(
                        src_ref=stage_ref.at[rs, cs],
                        dst_ref=recvx_ref.at[rs, cs],
                        send_sem=sx_send.at[ii, jj],
                        recv_sem=sx_recv.at[ii, jj],
                        device_id=(other_x, my_y),
                        device_id_type=pl.DeviceIdType.MESH,
                    ).wait_send()
                    if ii == _NOWN - 1 and jj == _NJ - 1:
                        pltpu.make_async_remote_copy(
                            src_ref=q_ref,
                            dst_ref=stk_ref.at[1, rs, cs],
                            send_sem=sy_send.at[ii, jj],
                            recv_sem=sy_recv.at[ii, jj],
                            device_id=(my_x, other_y),
                            device_id_type=pl.DeviceIdType.MESH,
                        ).wait_send()
                    pltpu.make_async_remote_copy(
                        src_ref=stk_ref.at[0, rs, cs],
                        dst_ref=stk_ref.at[1, rs, cs],
                        send_sem=sy_send.at[ii, jj],
                        recv_sem=sy_recv.at[ii, jj],
                        device_id=(my_x, other_y),
                        device_id_type=pl.DeviceIdType.MESH,
                    ).wait_recv()

    grid_spec = pltpu.PrefetchScalarGridSpec(
        num_scalar_prefetch=1,
        grid=(_NI, _NJ, _NK),
        in_specs=[
            pl.BlockSpec(
                (_BK, _BM), lambda i, j, k, s: (k, (i + 2 * (1 - s[0])) % 4)
            ),
            pl.BlockSpec((_BK, _BN), lambda i, j, k, s: (k, s[1] * _NJ + j)),
        ],
        out_specs=[
            pl.BlockSpec(memory_space=pl.ANY),
            pl.BlockSpec(memory_space=pl.ANY),
            pl.BlockSpec(memory_space=pl.ANY),
        ],
        scratch_shapes=[
            pltpu.VMEM((_BM, _BN), jnp.float32),
            pltpu.VMEM((_BM, _BN), jnp.bfloat16),
            pltpu.VMEM((_BM, _BN), jnp.bfloat16),
            pltpu.VMEM((_BM, _BN), jnp.bfloat16),
            pltpu.SemaphoreType.DMA((3,)),
            pltpu.SemaphoreType.DMA((_NOWN, _NJ)),
            pltpu.SemaphoreType.DMA((_NOWN, _NJ)),
            pltpu.SemaphoreType.DMA((_NOWN, _NJ)),
            pltpu.SemaphoreType.DMA((_NOWN, _NJ)),
        ],
    )
    stk, _, _ = pl.pallas_call(
        body,
        grid_spec=grid_spec,
        out_shape=[
            jax.ShapeDtypeStruct((2, M_HALF, N_HALF), jnp.bfloat16),
            jax.ShapeDtypeStruct((M_HALF, N_HALF), jnp.bfloat16),
            jax.ShapeDtypeStruct((M_HALF, N_HALF), jnp.bfloat16),
        ],
        compiler_params=pltpu.CompilerParams(
            dimension_semantics=("arbitrary", "arbitrary", "arbitrary"),
            vmem_limit_bytes=64 * 1024 * 1024,
            collective_id=0,
        ),
    )(scalars, x_shard, dy_shard)
    return stk


_CM2 = 512


def _assemble(scalars, stk):

    def body(s_ref, stk_ref, out_ref):
        out_ref[...] = stk_ref[0].astype(jnp.float32)

    grid_spec = pltpu.PrefetchScalarGridSpec(
        num_scalar_prefetch=1,
        grid=(2, M_HALF // _CM2),
        in_specs=[
            pl.BlockSpec((1, _CM2, N_HALF), lambda h, c, s: (h, c, 0)),
        ],
        out_specs=pl.BlockSpec(
            (_CM2, N_HALF),
            lambda h, c, s: (c, jnp.where(h == 0, s[1], 1 - s[1])),
        ),
    )
    return pl.pallas_call(
        body,
        grid_spec=grid_spec,
        out_shape=jax.ShapeDtypeStruct((M_HALF, N), jnp.float32),
        compiler_params=pltpu.CompilerParams(
            dimension_semantics=("arbitrary", "arbitrary"),
            vmem_limit_bytes=64 * 1024 * 1024,
        ),
    )(scalars, stk)


def kernel(x, dy):
    my_x = lax.axis_index("x")
    my_y = lax.axis_index("y")
    scalars = jnp.array([my_x, my_y], dtype=jnp.int32)
    stk = _fused(scalars, x, dy)
    return _assemble(scalars, stk)


# device time: 341263 ns/iter; 1.0125x vs baseline; 1.0125x over previous
import jax
import jax.numpy as jnp
from jax import lax
from jax.experimental import pallas as pl
from jax.experimental.pallas import tpu as pltpu

K = 4096
M = 4096
N = 8192
N_HALF = N // 2
M_HALF = M // 2

_BM, _BN, _BK = 1024, 1024, 2048
_NI = M // _BM
_NJ = N_HALF // _BN
_NK = K // _BK
_NOWN = _NI // 2


def _fused(scalars, x_shard, dy_shard):
    def body(
        s_ref,
        x_ref,
        dy_ref,
        stk_ref,
        recvx_ref,
        stage_ref,
        acc_ref,
        b_ref,
        q_ref,
        sendv_ref,
        copy_sems,
        sx_send,
        sx_recv,
        sy_send,
        sy_recv,
    ):
        i = pl.program_id(0)
        j = pl.program_id(1)
        k = pl.program_id(2)
        my_x = s_ref[0]
        my_y = s_ref[1]
        other_x = 1 - my_x
        other_y = 1 - my_y

        @pl.when((i == 0) & (j == 0) & (k == 0))
        def _():
            barrier = pltpu.get_barrier_semaphore()
            pl.semaphore_signal(
                barrier, inc=1, device_id=(other_x, my_y),
                device_id_type=pl.DeviceIdType.MESH,
            )
            pl.semaphore_signal(
                barrier, inc=1, device_id=(my_x, other_y),
                device_id_type=pl.DeviceIdType.MESH,
            )
            pl.semaphore_wait(barrier, 2)

        @pl.when(k == 0)
        def _():
            acc_ref[...] = jnp.zeros_like(acc_ref)

        acc_ref[...] += lax.dot_general(
            x_ref[...],
            dy_ref[...],
            dimension_numbers=(((0,), (0,)), ((), ())),
            preferred_element_type=jnp.float32,
        )

        @pl.when(k == _NK - 1)
        def _finalize():
            rows = pl.ds(i * _BM, _BM)
            rows_own = pl.ds((i - _NOWN) * _BM, _BM)
            cols = pl.ds(j * _BN, _BN)

            @pl.when(i < _NOWN)
            def _send_to_x_neighbor():
                sendv_ref[...] = acc_ref[...].astype(jnp.bfloat16)
                st = pltpu.make_async_copy(
                    sendv_ref, stage_ref.at[rows, cols], copy_sems.at[0]
                )
                st.start()
                st.wait()
                pltpu.make_async_remote_copy(
                    src_ref=stage_ref.at[rows, cols],
                    dst_ref=recvx_ref.at[rows, cols],
                    send_sem=sx_send.at[i, j],
                    recv_sem=sx_recv.at[i, j],
                    device_id=(other_x, my_y),
                    device_id_type=pl.DeviceIdType.MESH,
                ).start()

            @pl.when(i >= _NOWN)
            def _reduce_and_send_y():
                pltpu.make_async_remote_copy(
                    src_ref=stage_ref.at[rows_own, cols],
                    dst_ref=recvx_ref.at[rows_own, cols],
                    send_sem=sx_send.at[i - _NOWN, j],
                    recv_sem=sx_recv.at[i - _NOWN, j],
                    device_id=(other_x, my_y),
                    device_id_type=pl.DeviceIdType.MESH,
                ).wait_recv()
                ld = pltpu.make_async_copy(
                    recvx_ref.at[rows_own, cols], b_ref, copy_sems.at[1]
                )
                ld.start()
                ld.wait()
                q_ref[...] = (
                    acc_ref[...] + b_ref[...].astype(jnp.float32)
                ).astype(jnp.bfloat16)
                stq = pltpu.make_async_copy(
                    q_ref, stk_ref.at[0, rows_own, cols], copy_sems.at[2]
                )
                stq.start()
                stq.wait()
                pltpu.make_async_remote_copy(
                    src_ref=stk_ref.at[0, rows_own, cols],
                    dst_ref=stk_ref.at[1, rows_own, cols],
                    send_sem=sy_send.at[i - _NOWN, j],
                    recv_sem=sy_recv.at[i - _NOWN, j],
                    device_id=(my_x, other_y),
                    device_id_type=pl.DeviceIdType.MESH,
                ).start()

        @pl.when((i == _NI - 1) & (j == _NJ - 1) & (k == _NK - 1))
        def _drain():
            for ii in range(_NOWN):
                for jj in range(_NJ):
                    rs = pl.ds(ii * _BM, _BM)
                    cs = pl.ds(jj * _BN, _BN)
                    pltpu.make_async_remote_copy(
                        src_ref=stage_ref.at[rs, cs],
                        dst_ref=recvx_ref.at[rs, cs],
                        send_sem=sx_send.at[ii, jj],
                        recv_sem=sx_recv.at[ii, jj],
                        device_id=(other_x, my_y),
                        device_id_type=pl.DeviceIdType.MESH,
                    ).wait_send()
                    pltpu.make_async_remote_copy(
                        src_ref=stk_ref.at[0, rs, cs],
                        dst_ref=stk_ref.at[1, rs, cs],
                        send_sem=sy_send.at[ii, jj],
                        recv_sem=sy_recv.at[ii, jj],
                        device_id=(my_x, other_y),
                        device_id_type=pl.DeviceIdType.MESH,
                    ).wait_send()
                    pltpu.make_async_remote_copy(
                        src_ref=stk_ref.at[0, rs, cs],
                        dst_ref=stk_ref.at[1, rs, cs],
                        send_sem=sy_send.at[ii, jj],
                        recv_sem=sy_recv.at[ii, jj],
                        device_id=(my_x, other_y),
                        device_id_type=pl.DeviceIdType.MESH,
                    ).wait_recv()

    grid_spec = pltpu.PrefetchScalarGridSpec(
        num_scalar_prefetch=1,
        grid=(_NI, _NJ, _NK),
        in_specs=[
            pl.BlockSpec(
                (_BK, _BM), lambda i, j, k, s: (k, (i + 2 * (1 - s[0])) % 4)
            ),
            pl.BlockSpec((_BK, _BN), lambda i, j, k, s: (k, s[1] * _NJ + j)),
        ],
        out_specs=[
            pl.BlockSpec(memory_space=pl.ANY),
            pl.BlockSpec(memory_space=pl.ANY),
            pl.BlockSpec(memory_space=pl.ANY),
        ],
        scratch_shapes=[
            pltpu.VMEM((_BM, _BN), jnp.float32),
            pltpu.VMEM((_BM, _BN), jnp.bfloat16),
            pltpu.VMEM((_BM, _BN), jnp.bfloat16),
            pltpu.VMEM((_BM, _BN), jnp.bfloat16),
            pltpu.SemaphoreType.DMA((3,)),
            pltpu.SemaphoreType.DMA((_NOWN, _NJ)),
            pltpu.SemaphoreType.DMA((_NOWN, _NJ)),
            pltpu.SemaphoreType.DMA((_NOWN, _NJ)),
            pltpu.SemaphoreType.DMA((_NOWN, _NJ)),
        ],
    )
    stk, _, _ = pl.pallas_call(
        body,
        grid_spec=grid_spec,
        out_shape=[
            jax.ShapeDtypeStruct((2, M_HALF, N_HALF), jnp.bfloat16),
            jax.ShapeDtypeStruct((M_HALF, N_HALF), jnp.bfloat16),
            jax.ShapeDtypeStruct((M_HALF, N_HALF), jnp.bfloat16),
        ],
        compiler_params=pltpu.CompilerParams(
            dimension_semantics=("arbitrary", "arbitrary", "arbitrary"),
            vmem_limit_bytes=64 * 1024 * 1024,
            collective_id=0,
        ),
    )(scalars, x_shard, dy_shard)
    return stk


_CM2 = 512


def _assemble(scalars, stk):

    def body(s_ref, stk_ref, out_ref):
        out_ref[...] = stk_ref[0].astype(jnp.float32)

    grid_spec = pltpu.PrefetchScalarGridSpec(
        num_scalar_prefetch=1,
        grid=(2, M_HALF // _CM2),
        in_specs=[
            pl.BlockSpec((1, _CM2, N_HALF), lambda h, c, s: (h, c, 0)),
        ],
        out_specs=pl.BlockSpec(
            (_CM2, N_HALF),
            lambda h, c, s: (c, jnp.where(h == 0, s[1], 1 - s[1])),
        ),
    )
    return pl.pallas_call(
        body,
        grid_spec=grid_spec,
        out_shape=jax.ShapeDtypeStruct((M_HALF, N), jnp.float32),
        compiler_params=pltpu.CompilerParams(
            dimension_semantics=("arbitrary", "arbitrary"),
            vmem_limit_bytes=64 * 1024 * 1024,
        ),
    )(scalars, stk)


def kernel(x, dy):
    my_x = lax.axis_index("x")
    my_y = lax.axis_index("y")
    scalars = jnp.array([my_x, my_y], dtype=jnp.int32)
    stk = _fused(scalars, x, dy)
    return _assemble(scalars, stk)
